# baseline (device time: 57434 ns/iter reference)
import jax
import jax.numpy as jnp
from jax import lax
from jax.experimental import pallas as pl
from jax.experimental.pallas import tpu as pltpu

N_DEV = 4
B = 2
SQ = 256
SKV = 256
HQ = 4
DH = 64
DM = 512
BLK = 64


def kernel(x, Wq, K_ext, V_ext, Wo):
    x2 = x.reshape(B * SQ, DM)
    Wq_t = Wq.reshape(DM, HQ, DH).transpose(1, 0, 2)
    Wo_t = Wo.reshape(HQ, DH, DM)
    K_t = K_ext.transpose(2, 0, 3, 1)
    V_t = V_ext.transpose(2, 0, 1, 3)

    def body(x_ref, wq_ref, k_ref, v_ref, wo_ref, out_ref,
             wq_rbuf, wo_rbuf,
             wq_send_sems, wq_recv_sems, wo_send_sems, wo_recv_sems):
        my = lax.axis_index("i")

        barrier = pltpu.get_barrier_semaphore()
        for d in range(1, N_DEV):
            pl.semaphore_signal(
                barrier, inc=1,
                device_id=((my + d) % N_DEV,),
                device_id_type=pl.DeviceIdType.MESH,
            )
        pl.semaphore_wait(barrier, N_DEV - 1)

        sends = []
        for d in range(1, N_DEV):
            dst = (my + d) % N_DEV
            s = N_DEV - 1 - d
            for src, rbuf, ssems, rsems in (
                (wq_ref, wq_rbuf, wq_send_sems, wq_recv_sems),
                (wo_ref, wo_rbuf, wo_send_sems, wo_recv_sems),
            ):
                rdma = pltpu.make_async_remote_copy(
                    src_ref=src,
                    dst_ref=rbuf.at[s],
                    send_sem=ssems.at[d - 1],
                    recv_sem=rsems.at[s],
                    device_id=(dst,),
                    device_id_type=pl.DeviceIdType.MESH,
                )
                rdma.start()
                sends.append(rdma)

        mask = (
            lax.broadcasted_iota(jnp.int32, (SQ, SKV), 0) // BLK
            == lax.broadcasted_iota(jnp.int32, (SQ, SKV), 1) // BLK
        )
        xv = x_ref[...]

        def accum_term(wq_s, wo_s, j, init):
            for h in range(HQ):
                q_h = jnp.dot(xv, wq_s[h],
                              preferred_element_type=jnp.float32)
                gh = j * HQ + h
                for b in range(B):
                    q = q_h[b * SQ:(b + 1) * SQ, :]
                    scores = jnp.dot(q, k_ref[gh, b],
                                     preferred_element_type=jnp.float32) * 0.125
                    scores = jnp.where(mask, scores, -1e9)
                    m = jnp.max(scores, axis=-1, keepdims=True)
                    w = jnp.exp(scores - m)
                    w = w / jnp.sum(w, axis=-1, keepdims=True)
                    ctx = jnp.dot(w, v_ref[gh, b],
                                  preferred_element_type=jnp.float32)
                    contrib = jnp.dot(ctx, wo_s[h],
                                      preferred_element_type=jnp.float32)
                    if init and h == 0:
                        out_ref[b] = contrib
                    else:
                        out_ref[b] = out_ref[b] + contrib

        accum_term(wq_ref, wo_ref, my, init=True)

        for s in (0, 2, 1):
            for src, rbuf, ssems, rsems in (
                (wq_ref, wq_rbuf, wq_send_sems, wq_recv_sems),
                (wo_ref, wo_rbuf, wo_send_sems, wo_recv_sems),
            ):
                recv = pltpu.make_async_remote_copy(
                    src_ref=src,
                    dst_ref=rbuf.at[s],
                    send_sem=ssems.at[0],
                    recv_sem=rsems.at[s],
                    device_id=(my,),
                    device_id_type=pl.DeviceIdType.MESH,
                )
                recv.wait_recv()
            accum_term(wq_rbuf.at[s], wo_rbuf.at[s], (my + 1 + s) % N_DEV,
                       init=False)

        for rdma in sends:
            rdma.wait_send()

    out = pl.pallas_call(
        body,
        out_shape=jax.ShapeDtypeStruct((B, SQ, DM), jnp.float32),
        in_specs=[pl.BlockSpec(memory_space=pltpu.VMEM)] * 5,
        out_specs=pl.BlockSpec(memory_space=pltpu.VMEM),
        scratch_shapes=[
            pltpu.VMEM((N_DEV - 1, HQ, DM, DH), jnp.float32),
            pltpu.VMEM((N_DEV - 1, HQ, DH, DM), jnp.float32),
            pltpu.SemaphoreType.DMA((N_DEV - 1,)),
            pltpu.SemaphoreType.DMA((N_DEV - 1,)),
            pltpu.SemaphoreType.DMA((N_DEV - 1,)),
            pltpu.SemaphoreType.DMA((N_DEV - 1,)),
        ],
        compiler_params=pltpu.CompilerParams(collective_id=0),
    )(x2, Wq_t, K_t, V_t, Wo_t)
    return out


# device time: 38220 ns/iter; 1.5027x vs baseline; 1.5027x over previous
import jax
import jax.numpy as jnp
from jax import lax
from jax.experimental import pallas as pl
from jax.experimental.pallas import tpu as pltpu

N_DEV = 4
B = 2
SQ = 256
SKV = 256
HQ = 4
DH = 64
DM = 512
BLK = 64


def kernel(x, Wq, K_ext, V_ext, Wo):
    bf16 = jnp.bfloat16
    x2 = x.reshape(B * SQ, DM).astype(bf16)
    Wq_t = (Wq * 0.125).reshape(DM, HQ, DH).transpose(1, 0, 2).astype(bf16)
    Wo_t = Wo.reshape(HQ, DH, DM).astype(bf16)
    K_t = K_ext.transpose(2, 0, 3, 1).astype(bf16)
    V_t = V_ext.transpose(2, 0, 1, 3).astype(bf16)

    def body(x_ref, wq_ref, k_ref, v_ref, wo_ref, out_ref,
             wq_rbuf, wo_rbuf,
             wq_send_sems, wq_recv_sems, wo_send_sems, wo_recv_sems):
        my = lax.axis_index("i")

        barrier = pltpu.get_barrier_semaphore()
        for d in range(1, N_DEV):
            pl.semaphore_signal(
                barrier, inc=1,
                device_id=((my + d) % N_DEV,),
                device_id_type=pl.DeviceIdType.MESH,
            )
        pl.semaphore_wait(barrier, N_DEV - 1)

        sends = []
        for d in range(1, N_DEV):
            dst = (my + d) % N_DEV
            s = N_DEV - 1 - d
            for src, rbuf, ssems, rsems in (
                (wq_ref, wq_rbuf, wq_send_sems, wq_recv_sems),
                (wo_ref, wo_rbuf, wo_send_sems, wo_recv_sems),
            ):
                rdma = pltpu.make_async_remote_copy(
                    src_ref=src,
                    dst_ref=rbuf.at[s],
                    send_sem=ssems.at[d - 1],
                    recv_sem=rsems.at[s],
                    device_id=(dst,),
                    device_id_type=pl.DeviceIdType.MESH,
                )
                rdma.start()
                sends.append(rdma)

        maskf = (
            lax.broadcasted_iota(jnp.int32, (SQ, SKV), 0) // BLK
            == lax.broadcasted_iota(jnp.int32, (SQ, SKV), 1) // BLK
        ).astype(jnp.float32)
        xv = x_ref[...]

        def accum_term(wq_s, wo_s, j, init):
            for h in range(HQ):
                q_h = jnp.dot(xv, wq_s[h],
                              preferred_element_type=jnp.float32)
                q_h = q_h.astype(jnp.bfloat16)
                gh = j * HQ + h
                for b in range(B):
                    q = q_h[b * SQ:(b + 1) * SQ, :]
                    scores = jnp.dot(q, k_ref[gh, b],
                                     preferred_element_type=jnp.float32)
                    w = jnp.exp(scores) * maskf
                    r = jnp.sum(w, axis=-1, keepdims=True)
                    ctx = jnp.dot(w.astype(jnp.bfloat16), v_ref[gh, b],
                                  preferred_element_type=jnp.float32)
                    ctx = (ctx / r).astype(jnp.bfloat16)
                    contrib = jnp.dot(ctx, wo_s[h],
                                      preferred_element_type=jnp.float32)
                    if init and h == 0:
                        out_ref[b] = contrib
                    else:
                        out_ref[b] = out_ref[b] + contrib

        accum_term(wq_ref, wo_ref, my, init=True)

        for s in (0, 2, 1):
            for src, rbuf, ssems, rsems in (
                (wq_ref, wq_rbuf, wq_send_sems, wq_recv_sems),
                (wo_ref, wo_rbuf, wo_send_sems, wo_recv_sems),
            ):
                recv = pltpu.make_async_remote_copy(
                    src_ref=src,
                    dst_ref=rbuf.at[s],
                    send_sem=ssems.at[0],
                    recv_sem=rsems.at[s],
                    device_id=(my,),
                    device_id_type=pl.DeviceIdType.MESH,
                )
                recv.wait_recv()
            accum_term(wq_rbuf.at[s], wo_rbuf.at[s], (my + 1 + s) % N_DEV,
                       init=False)

        for rdma in sends:
            rdma.wait_send()

    out = pl.pallas_call(
        body,
        out_shape=jax.ShapeDtypeStruct((B, SQ, DM), jnp.float32),
        in_specs=[pl.BlockSpec(memory_space=pltpu.VMEM)] * 5,
        out_specs=pl.BlockSpec(memory_space=pltpu.VMEM),
        scratch_shapes=[
            pltpu.VMEM((N_DEV - 1, HQ, DM, DH), jnp.bfloat16),
            pltpu.VMEM((N_DEV - 1, HQ, DH, DM), jnp.bfloat16),
            pltpu.SemaphoreType.DMA((N_DEV - 1,)),
            pltpu.SemaphoreType.DMA((N_DEV - 1,)),
            pltpu.SemaphoreType.DMA((N_DEV - 1,)),
            pltpu.SemaphoreType.DMA((N_DEV - 1,)),
        ],
        compiler_params=pltpu.CompilerParams(collective_id=0),
    )(x2, Wq_t, K_t, V_t, Wo_t)
    return out


# device time: 29343 ns/iter; 1.9573x vs baseline; 1.3025x over previous
import jax
import jax.numpy as jnp
from jax import lax
from jax.experimental import pallas as pl
from jax.experimental.pallas import tpu as pltpu

N_DEV = 4
B = 2
SQ = 256
SKV = 256
HQ = 4
DH = 64
DM = 512
BLK = 64


def kernel(x, Wq, K_ext, V_ext, Wo):
    bf16 = jnp.bfloat16
    xT = x.reshape(B * SQ, DM).T.astype(bf16)
    WqT = (Wq * 0.125).T.astype(bf16)
    WoT = Wo.T.astype(bf16)
    K_n = K_ext.transpose(2, 0, 1, 3).astype(bf16)
    V_t = V_ext.transpose(2, 0, 3, 1).astype(bf16)

    def body(x_ref, wq_ref, k_ref, v_ref, wo_ref, out_ref,
             wq_rbuf, wo_rbuf,
             wq_send_sems, wq_recv_sems, wo_send_sems, wo_recv_sems):
        my = lax.axis_index("i")

        barrier = pltpu.get_barrier_semaphore()
        for d in range(1, N_DEV):
            pl.semaphore_signal(
                barrier, inc=1,
                device_id=((my + d) % N_DEV,),
                device_id_type=pl.DeviceIdType.MESH,
            )
        pl.semaphore_wait(barrier, N_DEV - 1)

        sends = []
        for d in range(1, N_DEV):
            dst = (my + d) % N_DEV
            s = N_DEV - 1 - d
            for src, rbuf, ssems, rsems in (
                (wq_ref, wq_rbuf, wq_send_sems, wq_recv_sems),
                (wo_ref, wo_rbuf, wo_send_sems, wo_recv_sems),
            ):
                rdma = pltpu.make_async_remote_copy(
                    src_ref=src,
                    dst_ref=rbuf.at[s],
                    send_sem=ssems.at[d - 1],
                    recv_sem=rsems.at[s],
                    device_id=(dst,),
                    device_id_type=pl.DeviceIdType.MESH,
                )
                rdma.start()
                sends.append(rdma)

        maskf = (
            lax.broadcasted_iota(jnp.int32, (SKV, SQ), 0) // BLK
            == lax.broadcasted_iota(jnp.int32, (SKV, SQ), 1) // BLK
        ).astype(jnp.float32)
        xv = x_ref[...]

        def accum_term(wq_s, wo_s, j, init):
            qT = jnp.dot(wq_s[...], xv,
                         preferred_element_type=jnp.float32)
            qT = qT.astype(jnp.bfloat16)
            for b in range(B):
                ctxT = []
                for h in range(HQ):
                    q_bh = qT[h * DH:(h + 1) * DH, b * SQ:(b + 1) * SQ]
                    sT = jnp.dot(k_ref[j * HQ + h, b], q_bh,
                                 preferred_element_type=jnp.float32)
                    w = jnp.exp(sT) * maskf
                    r = jnp.sum(w, axis=0, keepdims=True)
                    cT = jnp.dot(v_ref[j * HQ + h, b], w.astype(jnp.bfloat16),
                                 preferred_element_type=jnp.float32)
                    ctxT.append((cT / r).astype(jnp.bfloat16))
                ctxT = jnp.concatenate(ctxT, axis=0)
                contribT = jnp.dot(wo_s[...], ctxT,
                                   preferred_element_type=jnp.float32)
                if init:
                    out_ref[b] = contribT
                else:
                    out_ref[b] = out_ref[b] + contribT

        accum_term(wq_ref, wo_ref, my, init=True)

        for s in (0, 2, 1):
            for src, rbuf, ssems, rsems in (
                (wq_ref, wq_rbuf, wq_send_sems, wq_recv_sems),
                (wo_ref, wo_rbuf, wo_send_sems, wo_recv_sems),
            ):
                recv = pltpu.make_async_remote_copy(
                    src_ref=src,
                    dst_ref=rbuf.at[s],
                    send_sem=ssems.at[0],
                    recv_sem=rsems.at[s],
                    device_id=(my,),
                    device_id_type=pl.DeviceIdType.MESH,
                )
                recv.wait_recv()
            accum_term(wq_rbuf.at[s], wo_rbuf.at[s], (my + 1 + s) % N_DEV,
                       init=False)

        for rdma in sends:
            rdma.wait_send()

    outT = pl.pallas_call(
        body,
        out_shape=jax.ShapeDtypeStruct((B, DM, SQ), jnp.float32),
        in_specs=[pl.BlockSpec(memory_space=pltpu.VMEM)] * 5,
        out_specs=pl.BlockSpec(memory_space=pltpu.VMEM),
        scratch_shapes=[
            pltpu.VMEM((N_DEV - 1, HQ * DH, DM), jnp.bfloat16),
            pltpu.VMEM((N_DEV - 1, DM, HQ * DH), jnp.bfloat16),
            pltpu.SemaphoreType.DMA((N_DEV - 1,)),
            pltpu.SemaphoreType.DMA((N_DEV - 1,)),
            pltpu.SemaphoreType.DMA((N_DEV - 1,)),
            pltpu.SemaphoreType.DMA((N_DEV - 1,)),
        ],
        compiler_params=pltpu.CompilerParams(collective_id=0),
    )(xT, WqT, K_n, V_t, WoT)
    return outT.transpose(0, 2, 1)


# device time: 22632 ns/iter; 2.5377x vs baseline; 1.2965x over previous
import jax
import jax.numpy as jnp
from jax import lax
from jax.experimental import pallas as pl
from jax.experimental.pallas import tpu as pltpu

N_DEV = 4
B = 2
SQ = 256
SKV = 256
HQ = 4
DH = 64
DM = 512
BLK = 64


def kernel(x, Wq, K_ext, V_ext, Wo):
    bf16 = jnp.bfloat16
    x2 = x.reshape(B * SQ, DM)
    WqT = (Wq * 0.125).T.astype(bf16)
    Wo_b = Wo.astype(bf16)
    K_n = K_ext.transpose(2, 0, 1, 3).astype(bf16)
    V_t = V_ext.transpose(2, 0, 3, 1).astype(bf16)

    def body(x_ref, wq_ref, k_ref, v_ref, wo_ref, out_ref,
             wq_rbuf, wo_rbuf,
             wq_send_sems, wq_recv_sems, wo_send_sems, wo_recv_sems):
        my = lax.axis_index("i")

        barrier = pltpu.get_barrier_semaphore()
        for d in range(1, N_DEV):
            pl.semaphore_signal(
                barrier, inc=1,
                device_id=((my + d) % N_DEV,),
                device_id_type=pl.DeviceIdType.MESH,
            )
        pl.semaphore_wait(barrier, N_DEV - 1)

        sends = []
        for src, rbuf, ssems, rsems in (
            (wq_ref, wq_rbuf, wq_send_sems, wq_recv_sems),
            (wo_ref, wo_rbuf, wo_send_sems, wo_recv_sems),
        ):
            for d in range(1, N_DEV):
                rdma = pltpu.make_async_remote_copy(
                    src_ref=src,
                    dst_ref=rbuf.at[N_DEV - 1 - d],
                    send_sem=ssems.at[d - 1],
                    recv_sem=rsems.at[N_DEV - 1 - d],
                    device_id=((my + d) % N_DEV,),
                    device_id_type=pl.DeviceIdType.MESH,
                )
                rdma.start()
                sends.append(rdma)

        def wait_recv(rbuf, ssems, rsems, s):
            pltpu.make_async_remote_copy(
                src_ref=rbuf.at[s],
                dst_ref=rbuf.at[s],
                send_sem=ssems.at[0],
                recv_sem=rsems.at[s],
                device_id=(my,),
                device_id_type=pl.DeviceIdType.MESH,
            ).wait_recv()

        maskf = (
            lax.broadcasted_iota(jnp.int32, (SKV, SQ), 0) // BLK
            == lax.broadcasted_iota(jnp.int32, (SKV, SQ), 1) // BLK
        ).astype(jnp.float32)
        xv = x_ref[...].astype(bf16)

        def attn_stage(wq_s, j):
            qT = lax.dot_general(wq_s[...], xv, (((1,), (1,)), ((), ())),
                                 preferred_element_type=jnp.float32)
            qT = qT.astype(bf16)
            cts = []
            for b in range(B):
                ctxT = []
                for h in range(HQ):
                    gh = j * HQ + h
                    q_bh = qT[h * DH:(h + 1) * DH, b * SQ:(b + 1) * SQ]
                    sT = jnp.dot(k_ref[gh, b], q_bh,
                                 preferred_element_type=jnp.float32)
                    w = jnp.exp(sT) * maskf
                    r = jnp.sum(w, axis=0, keepdims=True)
                    cT = jnp.dot(v_ref[gh, b], w.astype(bf16),
                                 preferred_element_type=jnp.float32)
                    ctxT.append((cT / r).astype(bf16))
                cts.append(jnp.concatenate(ctxT, axis=0))
            return cts

        def out_stage(wo_s, cts, init):
            for b in range(B):
                contrib = lax.dot_general(cts[b], wo_s[...],
                                          (((0,), (0,)), ((), ())),
                                          preferred_element_type=jnp.float32)
                if init:
                    out_ref[b] = contrib
                else:
                    out_ref[b] = out_ref[b] + contrib

        out_stage(wo_ref, attn_stage(wq_ref, my), init=True)

        order = (2, 1, 0)
        cts_all = {}
        for s in order:
            wait_recv(wq_rbuf, wq_send_sems, wq_recv_sems, s)
            cts_all[s] = attn_stage(wq_rbuf.at[s], (my + 1 + s) % N_DEV)
        for s in order:
            wait_recv(wo_rbuf, wo_send_sems, wo_recv_sems, s)
            out_stage(wo_rbuf.at[s], cts_all[s], init=False)

        for rdma in sends:
            rdma.wait_send()

    return pl.pallas_call(
        body,
        out_shape=jax.ShapeDtypeStruct((B, SQ, DM), jnp.float32),
        in_specs=[pl.BlockSpec(memory_space=pltpu.VMEM)] * 5,
        out_specs=pl.BlockSpec(memory_space=pltpu.VMEM),
        scratch_shapes=[
            pltpu.VMEM((N_DEV - 1, HQ * DH, DM), jnp.bfloat16),
            pltpu.VMEM((N_DEV - 1, HQ * DH, DM), jnp.bfloat16),
            pltpu.SemaphoreType.DMA((N_DEV - 1,)),
            pltpu.SemaphoreType.DMA((N_DEV - 1,)),
            pltpu.SemaphoreType.DMA((N_DEV - 1,)),
            pltpu.SemaphoreType.DMA((N_DEV - 1,)),
        ],
        compiler_params=pltpu.CompilerParams(collective_id=0),
    )(x2, WqT, K_n, V_t, Wo_b)
